# baseline (speedup 1.0000x reference)
import functools

import jax
import jax.numpy as jnp
from jax import lax
from jax.experimental import pallas as pl
from jax.experimental.pallas import tpu as pltpu
from jax.experimental.pallas import tpu_sc as plsc

_L = 16
_S = 4096
_ROW = 50
_M = 1000000

_mesh = plsc.VectorSubcoreMesh(core_axis_name="c", subcore_axis_name="s")


def _log_f32(x):
    xi = plsc.bitcast(x, jnp.int32)
    e = (xi >> 23) - 127
    m = plsc.bitcast((xi & 0x7FFFFF) | (127 << 23), jnp.float32)
    t = (m - 1.0) / (m + 1.0)
    t2 = t * t
    poly = 1.0 + t2 * (1.0 / 3.0 + t2 * (1.0 / 5.0 + t2 * (1.0 / 7.0 + t2 / 9.0)))
    ln_m = 2.0 * t * poly
    return e.astype(jnp.float32) * 0.6931471805599453 + ln_m


@functools.partial(
    pl.kernel,
    out_type=(
        jax.ShapeDtypeStruct((_L,), jnp.int32),
        jax.ShapeDtypeStruct((_ROW, _M), jnp.int32),
        jax.ShapeDtypeStruct((_M,), jnp.int32),
        jax.ShapeDtypeStruct((_M,), jnp.float32),
    ),
    mesh=_mesh,
    compiler_params=pltpu.CompilerParams(needs_layout_passes=False),
    scratch_types=[
        pltpu.VMEM((_L,), jnp.int32),
        pltpu.VMEM((_ROW, _L), jnp.int32),
        pltpu.VMEM((_L,), jnp.int32),
        pltpu.VMEM((_S,), jnp.float32),
        pltpu.VMEM((_L,), jnp.float32),
        pltpu.VMEM((_L,), jnp.int32),
        pltpu.VMEM((_L,), jnp.float32),
        pltpu.VMEM((_ROW, 128), jnp.int32),
        pltpu.SemaphoreType.DMA,
        pltpu.SemaphoreType.DMA,
        pltpu.SemaphoreType.DMA,
    ],
)
def _sc_add(size_hbm, nsl_hbm, nseq_hbm, nlsd_hbm, seq_in, len_in, lp_in,
            out_size_hbm, seq_out, len_out, lp_out,
            vs, vrow, vn, vx, vlp, vwl, vwp, vw, sem_s, sem_l, sem_p):
    c = lax.axis_index("c")
    s = lax.axis_index("s")

    @pl.when(jnp.logical_and(c == 0, s == 0))
    def _():
        big = pltpu.async_copy(seq_in, seq_out, sem_s)
        cpl = pltpu.async_copy(len_in, len_out, sem_l)
        cpp = pltpu.async_copy(lp_in, lp_out, sem_p)

        pltpu.sync_copy(size_hbm, vs)
        pltpu.sync_copy(nsl_hbm, vn)
        pltpu.sync_copy(nseq_hbm, vrow)
        pltpu.sync_copy(nlsd_hbm, vx)

        idx = lax.reduce_max(vs[...], axes=(0,))

        base = pl.multiple_of(jnp.minimum((idx >> 3) << 3, _M - _L), 8)
        off = idx - base
        lane = lax.iota(jnp.int32, _L)
        hit = lane == off

        base128 = pl.multiple_of((idx >> 7) << 7, 128)
        off128 = idx - base128
        chunk = pl.multiple_of((off128 >> 4) << 4, 16)
        hit16 = (lane + chunk) == off128
        pltpu.sync_copy(seq_in.at[:, pl.ds(base128, 128)], vw)
        for j in range(_ROW):
            vw[j, pl.ds(chunk, _L)] = jnp.where(
                hit16, vrow[j, :], vw[j, pl.ds(chunk, _L)])

        pltpu.sync_copy(len_in.at[pl.ds(base, _L)], vwl)
        vwl[...] = jnp.where(hit, vn[...], vwl[...])

        pltpu.sync_copy(lp_in.at[pl.ds(base, _L)], vwp)

        def max_body(i, acc):
            return jnp.maximum(acc, vx[pl.ds(i * _L, _L)])

        mvec = lax.fori_loop(1, _S // _L, max_body, vx[pl.ds(0, _L)])
        mmax = jnp.full((_L,), jnp.max(mvec))

        def sum_body(i, acc):
            return acc + jnp.exp(vx[pl.ds(i * _L, _L)] - mmax)

        svec = lax.fori_loop(0, _S // _L, sum_body, jnp.zeros((_L,), jnp.float32))
        tot = jnp.full((_L,), jnp.sum(svec))
        vlp[...] = mmax + _log_f32(tot)
        vwp[...] = jnp.where(hit, vlp[...], vwp[...])

        vs[...] = vs[...] + 1
        pltpu.sync_copy(vs, out_size_hbm)

        cpl.wait()
        pltpu.sync_copy(vwl, len_out.at[pl.ds(base, _L)])
        cpp.wait()
        pltpu.sync_copy(vwp, lp_out.at[pl.ds(base, _L)])
        big.wait()
        pltpu.sync_copy(vw, seq_out.at[:, pl.ds(base128, 128)])


def kernel(sequences, sequence_lengths, log_probabilities, size,
           node_sequence, node_sequence_length, node_log_state_distribution):
    size16 = jnp.broadcast_to(jnp.asarray(size, jnp.int32), (_L,))
    nsl16 = jnp.broadcast_to(jnp.asarray(node_sequence_length, jnp.int32), (_L,))
    nseq_b = jnp.broadcast_to(
        jnp.asarray(node_sequence, jnp.int32)[:, None], (_ROW, _L))

    out16, seq_t, lens, lps = _sc_add(
        size16, nsl16, nseq_b, node_log_state_distribution,
        sequences.T, sequence_lengths, log_probabilities)

    return seq_t.T, lens, lps, out16[0]


# speedup vs baseline: 40.1182x; 40.1182x over previous
import functools

import jax
import jax.numpy as jnp
from jax import lax
from jax.experimental import pallas as pl
from jax.experimental.pallas import tpu as pltpu
from jax.experimental.pallas import tpu_sc as plsc

_L = 16
_S = 4096
_ROW = 50
_M = 1000000

_mesh = plsc.VectorSubcoreMesh(core_axis_name="c", subcore_axis_name="s")


def _log_f32(x):
    xi = plsc.bitcast(x, jnp.int32)
    e = (xi >> 23) - 127
    m = plsc.bitcast((xi & 0x7FFFFF) | (127 << 23), jnp.float32)
    t = (m - 1.0) / (m + 1.0)
    t2 = t * t
    poly = 1.0 + t2 * (1.0 / 3.0 + t2 * (1.0 / 5.0 + t2 * (1.0 / 7.0 + t2 / 9.0)))
    ln_m = 2.0 * t * poly
    return e.astype(jnp.float32) * 0.6931471805599453 + ln_m


def _first_tile():
    return jnp.logical_and(lax.axis_index("c") == 0, lax.axis_index("s") == 0)


@functools.partial(
    pl.kernel,
    out_type=jax.ShapeDtypeStruct((_L,), jnp.int32),
    mesh=_mesh,
    compiler_params=pltpu.CompilerParams(needs_layout_passes=False),
    scratch_types=[
        pltpu.VMEM((_L,), jnp.int32),
        pltpu.VMEM((_L,), jnp.int32),
        pltpu.VMEM((_S,), jnp.float32),
        pltpu.VMEM((_L,), jnp.float32),
        pltpu.VMEM((_L,), jnp.int32),
        pltpu.VMEM((_L,), jnp.float32),
    ],
)
def _sc_scalars(size_hbm, nsl_hbm, nlsd_hbm, len_ref, lp_ref, out_size_hbm,
                vs, vn, vx, vlp, vwl, vwp):
    @pl.when(_first_tile())
    def _():
        pltpu.sync_copy(size_hbm, vs)
        pltpu.sync_copy(nsl_hbm, vn)
        pltpu.sync_copy(nlsd_hbm, vx)

        idx = lax.reduce_max(vs[...], axes=(0,))

        base = pl.multiple_of(jnp.minimum((idx >> 3) << 3, _M - _L), 8)
        off = idx - base
        hit = lax.iota(jnp.int32, _L) == off

        pltpu.sync_copy(len_ref.at[pl.ds(base, _L)], vwl)
        vwl[...] = jnp.where(hit, vn[...], vwl[...])
        pltpu.sync_copy(vwl, len_ref.at[pl.ds(base, _L)])

        def max_body(i, acc):
            return jnp.maximum(acc, vx[pl.ds(i * _L, _L)])

        mvec = lax.fori_loop(1, _S // _L, max_body, vx[pl.ds(0, _L)])
        mmax = jnp.full((_L,), jnp.max(mvec))

        def sum_body(i, acc):
            return acc + jnp.exp(vx[pl.ds(i * _L, _L)] - mmax)

        svec = lax.fori_loop(0, _S // _L, sum_body, jnp.zeros((_L,), jnp.float32))
        tot = jnp.full((_L,), jnp.sum(svec))
        vlp[...] = mmax + _log_f32(tot)

        pltpu.sync_copy(lp_ref.at[pl.ds(base, _L)], vwp)
        vwp[...] = jnp.where(hit, vlp[...], vwp[...])
        pltpu.sync_copy(vwp, lp_ref.at[pl.ds(base, _L)])

        vs[...] = vs[...] + 1
        pltpu.sync_copy(vs, out_size_hbm)


@functools.partial(
    pl.kernel,
    out_type=(),
    mesh=_mesh,
    compiler_params=pltpu.CompilerParams(needs_layout_passes=False),
    scratch_types=[
        pltpu.VMEM((_L,), jnp.int32),
        pltpu.VMEM((_ROW, _L), jnp.int32),
        pltpu.VMEM((_ROW, 128), jnp.int32),
    ],
)
def _sc_seq(size_hbm, nseq_hbm, seq_ref, vs, vrow, vw):
    @pl.when(_first_tile())
    def _():
        pltpu.sync_copy(size_hbm, vs)
        pltpu.sync_copy(nseq_hbm, vrow)

        idx = lax.reduce_max(vs[...], axes=(0,))

        base128 = pl.multiple_of((idx >> 7) << 7, 128)
        off128 = idx - base128
        chunk = pl.multiple_of((off128 >> 4) << 4, 16)
        hit16 = (lax.iota(jnp.int32, _L) + chunk) == off128
        pltpu.sync_copy(seq_ref.at[:, pl.ds(base128, 128)], vw)
        for j in range(_ROW):
            vw[j, pl.ds(chunk, _L)] = jnp.where(
                hit16, vrow[j, :], vw[j, pl.ds(chunk, _L)])
        pltpu.sync_copy(vw, seq_ref.at[:, pl.ds(base128, 128)])


def kernel(sequences, sequence_lengths, log_probabilities, size,
           node_sequence, node_sequence_length, node_log_state_distribution):
    size16 = jnp.broadcast_to(jnp.asarray(size, jnp.int32), (_L,))
    nsl16 = jnp.broadcast_to(jnp.asarray(node_sequence_length, jnp.int32), (_L,))
    nseq_b = jnp.broadcast_to(
        jnp.asarray(node_sequence, jnp.int32)[:, None], (_ROW, _L))

    seq_ref = jax.new_ref(sequences.T)
    len_ref = jax.new_ref(sequence_lengths)
    lp_ref = jax.new_ref(log_probabilities)

    out16 = _sc_scalars(size16, nsl16, node_log_state_distribution,
                        len_ref, lp_ref)
    _sc_seq(size16, nseq_b, seq_ref)

    return seq_ref[...].T, len_ref[...], lp_ref[...], out16[0]


# speedup vs baseline: 40.9054x; 1.0196x over previous
import functools

import jax
import jax.numpy as jnp
from jax import lax
from jax.experimental import pallas as pl
from jax.experimental.pallas import tpu as pltpu
from jax.experimental.pallas import tpu_sc as plsc

_L = 16
_S = 4096
_ROW = 50
_M = 1000000

_mesh = plsc.VectorSubcoreMesh(core_axis_name="c", subcore_axis_name="s")


def _log_f32(x):
    xi = plsc.bitcast(x, jnp.int32)
    e = (xi >> 23) - 127
    m = plsc.bitcast((xi & 0x7FFFFF) | (127 << 23), jnp.float32)
    t = (m - 1.0) / (m + 1.0)
    t2 = t * t
    poly = 1.0 + t2 * (1.0 / 3.0 + t2 * (1.0 / 5.0 + t2 * (1.0 / 7.0 + t2 / 9.0)))
    ln_m = 2.0 * t * poly
    return e.astype(jnp.float32) * 0.6931471805599453 + ln_m


@functools.partial(
    pl.kernel,
    out_type=(
        jax.ShapeDtypeStruct((_L,), jnp.int32),
        jax.ShapeDtypeStruct((_ROW, 128), jnp.int32),
    ),
    mesh=_mesh,
    compiler_params=pltpu.CompilerParams(needs_layout_passes=False),
    scratch_types=[
        pltpu.VMEM((_L,), jnp.int32),
        pltpu.VMEM((_ROW, _L), jnp.int32),
        pltpu.VMEM((_S,), jnp.float32),
        pltpu.VMEM((_L,), jnp.float32),
        pltpu.VMEM((_L,), jnp.int32),
        pltpu.VMEM((_L,), jnp.float32),
        pltpu.VMEM((_ROW, 128), jnp.int32),
    ],
)
def _sc_add(combo_hbm, nseq_hbm, nlsd_hbm, seq_in, len_ref, lp_ref,
            out_size_hbm, win_hbm,
            vc, vrow, vx, vlp, vwl, vwp, vw):
    @pl.when(jnp.logical_and(lax.axis_index("c") == 0,
                             lax.axis_index("s") == 0))
    def _():
        pltpu.sync_copy(combo_hbm, vc)
        pltpu.sync_copy(nseq_hbm, vrow)
        pltpu.sync_copy(nlsd_hbm, vx)

        lane = lax.iota(jnp.int32, _L)
        v = vc[...]
        zero = jnp.zeros((_L,), jnp.int32)
        idx = lax.reduce_max(jnp.where(lane < 8, v, zero), axes=(0,))
        vn = jnp.full((_L,), lax.reduce_max(jnp.where(lane >= 8, v, zero),
                                            axes=(0,)))

        base128 = pl.multiple_of((idx >> 7) << 7, 128)
        off128 = idx - base128
        chunk = pl.multiple_of((off128 >> 4) << 4, 16)
        hit16 = (lane + chunk) == off128
        pltpu.sync_copy(seq_in.at[:, pl.ds(base128, 128)], vw)
        for j in range(_ROW):
            vw[j, pl.ds(chunk, _L)] = jnp.where(
                hit16, vrow[j, :], vw[j, pl.ds(chunk, _L)])
        pltpu.sync_copy(vw, win_hbm)

        base = pl.multiple_of(jnp.minimum((idx >> 3) << 3, _M - _L), 8)
        hit = lane == (idx - base)

        pltpu.sync_copy(len_ref.at[pl.ds(base, _L)], vwl)
        vwl[...] = jnp.where(hit, vn, vwl[...])
        pltpu.sync_copy(vwl, len_ref.at[pl.ds(base, _L)])

        def max_body(i, acc):
            return jnp.maximum(acc, vx[pl.ds(i * _L, _L)])

        mvec = lax.fori_loop(1, _S // _L, max_body, vx[pl.ds(0, _L)])
        mmax = jnp.full((_L,), jnp.max(mvec))

        def sum_body(i, acc):
            return acc + jnp.exp(vx[pl.ds(i * _L, _L)] - mmax)

        svec = lax.fori_loop(0, _S // _L, sum_body, jnp.zeros((_L,), jnp.float32))
        tot = jnp.full((_L,), jnp.sum(svec))
        vlp[...] = mmax + _log_f32(tot)

        pltpu.sync_copy(lp_ref.at[pl.ds(base, _L)], vwp)
        vwp[...] = jnp.where(hit, vlp[...], vwp[...])
        pltpu.sync_copy(vwp, lp_ref.at[pl.ds(base, _L)])

        vc[...] = jnp.where(lane < 8, v + 1, v + 1)
        pltpu.sync_copy(vc, out_size_hbm)


def _paste_body(size_ref, win_ref, seq_any, out_ref):
    del size_ref, seq_any
    out_ref[...] = win_ref[...]


def _paste(size1, win, seq_t):
    return pl.pallas_call(
        _paste_body,
        grid_spec=pltpu.PrefetchScalarGridSpec(
            num_scalar_prefetch=1,
            grid=(1,),
            in_specs=[
                pl.BlockSpec((_ROW, 128), lambda i, sref: (0, 0)),
                pl.BlockSpec(memory_space=pl.ANY),
            ],
            out_specs=pl.BlockSpec(
                (_ROW, 128), lambda i, sref: (0, sref[0] // 128)),
        ),
        out_shape=jax.ShapeDtypeStruct((_ROW, _M), jnp.int32),
        input_output_aliases={2: 0},
    )(size1, win, seq_t)


def kernel(sequences, sequence_lengths, log_probabilities, size,
           node_sequence, node_sequence_length, node_log_state_distribution):
    size_i = jnp.asarray(size, jnp.int32)
    nsl_i = jnp.asarray(node_sequence_length, jnp.int32)
    combo = jnp.where(jnp.arange(_L) < 8, size_i, nsl_i)
    nseq_b = jnp.broadcast_to(
        jnp.asarray(node_sequence, jnp.int32)[:, None], (_ROW, _L))
    size1 = size_i.reshape(1)

    seq_t = sequences.T
    len_ref = jax.new_ref(sequence_lengths)
    lp_ref = jax.new_ref(log_probabilities)

    out16, win = _sc_add(combo, nseq_b, node_log_state_distribution,
                         seq_t, len_ref, lp_ref)
    seq_new_t = _paste(size1, win, seq_t)

    return seq_new_t.T, len_ref[...], lp_ref[...], out16[0]
